# baseline (speedup 1.0000x reference)
import functools

import jax
import jax.numpy as jnp
from jax import lax
from jax.experimental import pallas as pl
from jax.experimental.pallas import tpu as pltpu
from jax.experimental.pallas import tpu_sc as plsc

N = 10000
NPAD = 10240
F = 128
E = 320000
NC = 2
NS = 16
NW = NC * NS
CHUNK = 128
GC = 1
AGG_CHUNK = GC * CHUNK
NCHUNK = 80
PHASES = 5
PCH = NCHUNK // PHASES
EPT = NCHUNK * AGG_CHUNK
EPAD = EPT * NW
ROWS_PER_TILE = NPAD // NS

_sc_mesh = plsc.VectorSubcoreMesh(core_axis_name="c", subcore_axis_name="s")


@functools.partial(
    pl.kernel,
    out_type=jax.ShapeDtypeStruct((NW, NPAD), jnp.float32),
    mesh=_sc_mesh,
    scratch_types=[
        pltpu.VMEM((NCHUNK, CHUNK), jnp.int32),
        pltpu.VMEM((NPAD,), jnp.float32),
    ],
    compiler_params=pltpu.CompilerParams(needs_layout_passes=False),
)
def _deg_kernel(sd_hbm, out_hbm, didx_v, deg_v):
    c = lax.axis_index("c")
    s = lax.axis_index("s")
    w = c * NS + s
    zeros16 = jnp.zeros((16,), jnp.float32)

    @pl.loop(0, NPAD // 16)
    def _(i):
        deg_v[pl.ds(i * 16, 16)] = zeros16

    pltpu.sync_copy(sd_hbm.at[1, w], didx_v)
    ones16 = jnp.ones((16,), jnp.float32)

    @pl.loop(0, NCHUNK)
    def _(i):
        for j in range(CHUNK // 16):
            idx = didx_v[i, pl.ds(j * 16, 16)]
            plsc.addupdate_scatter(deg_v, [idx], ones16)

    pltpu.sync_copy(deg_v, out_hbm.at[w])


NBUF = 2


@functools.partial(
    pl.kernel,
    out_type=jax.ShapeDtypeStruct((NC, NPAD, F), jnp.float32),
    mesh=_sc_mesh,
    scratch_types=[
        pltpu.VMEM((PCH, CHUNK), jnp.int32),
        pltpu.VMEM((PCH, CHUNK), jnp.int32),
        pltpu.VMEM((AGG_CHUNK, F), jnp.float32),
        pltpu.VMEM((AGG_CHUNK, F), jnp.float32),
        pltpu.VMEM_SHARED((NPAD, F), jnp.float32),
        pltpu.SemaphoreType.DMA,
        pltpu.SemaphoreType.DMA,
        pltpu.SemaphoreType.DMA,
        pltpu.SemaphoreType.DMA,
    ],
)
def _agg_kernel(y_hbm, sd_hbm, out_hbm, sidx_v, didx_v, rows0_v,
                rows1_v, acc_sh, g0, g1, s0, s1):
    c = lax.axis_index("c")
    s = lax.axis_index("s")
    w = c * NS + s
    rows = (rows0_v, rows1_v)
    gsem = (g0, g1)
    ssem = (s0, s1)
    zeros16 = jnp.zeros((16,), jnp.float32)

    @pl.loop(0, AGG_CHUNK)
    def _(i):
        for j in range(F // 16):
            rows0_v[i, pl.ds(j * 16, 16)] = zeros16

    for r in range(ROWS_PER_TILE // AGG_CHUNK):
        pltpu.sync_copy(
            rows0_v.at[pl.ds(0, AGG_CHUNK)],
            acc_sh.at[pl.ds(s * ROWS_PER_TILE + r * AGG_CHUNK, AGG_CHUNK)])
    plsc.subcore_barrier()

    def gather(i, b):
        pltpu.async_copy(y_hbm.at[sidx_v.at[i]], rows[b], gsem[b])

    def wait_gather(i, b):
        pltpu.make_async_copy(y_hbm.at[sidx_v.at[i]], rows[b], gsem[b]).wait()

    def scatter(i, b):
        pltpu.async_copy(rows[b], acc_sh.at[didx_v.at[i]], ssem[b], add=True)

    def wait_scatter(i, b):
        pltpu.make_async_copy(rows[b], acc_sh.at[didx_v.at[i]],
                              ssem[b]).wait()

    for h in range(PHASES):
        pltpu.sync_copy(sd_hbm.at[0, w, pl.ds(h * PCH, PCH)], sidx_v)
        pltpu.sync_copy(sd_hbm.at[1, w, pl.ds(h * PCH, PCH)], didx_v)

        for b in range(NBUF):
            gather(b, b)

        @pl.loop(0, PCH, step=NBUF)
        def _(i):
            for b in range(NBUF):
                wait_gather(i + b, b)
                scatter(i + b, b)

                @pl.when(i + b + NBUF < PCH)
                def _():
                    wait_scatter(i + b, b)
                    gather(i + b + NBUF, b)

        for b in range(NBUF):
            wait_scatter(PCH - NBUF + b, b)

    plsc.subcore_barrier()
    pltpu.sync_copy(acc_sh.at[pl.ds(s * ROWS_PER_TILE, ROWS_PER_TILE)],
                    out_hbm.at[c, pl.ds(s * ROWS_PER_TILE, ROWS_PER_TILE)])


BLK = 1024


def _mm_first_body(x_ref, w_ref, degt_ref, y_ref, dinv_ref):
    deg = jnp.sum(degt_ref[...], axis=1, keepdims=True)
    dinv = lax.rsqrt(1.0 + deg)
    y_ref[...] = jnp.dot(x_ref[...], w_ref[...],
                         preferred_element_type=jnp.float32) * dinv
    dinv_ref[...] = dinv


_mm_first = pl.pallas_call(
    _mm_first_body,
    grid=(NPAD // BLK,),
    in_specs=[
        pl.BlockSpec((BLK, F), lambda i: (i, 0)),
        pl.BlockSpec((F, F), lambda i: (0, 0)),
        pl.BlockSpec((BLK, NW), lambda i: (i, 0)),
    ],
    out_specs=[
        pl.BlockSpec((BLK, F), lambda i: (i, 0)),
        pl.BlockSpec((BLK, 1), lambda i: (i, 0)),
    ],
    out_shape=[
        jax.ShapeDtypeStruct((NPAD, F), jnp.float32),
        jax.ShapeDtypeStruct((NPAD, 1), jnp.float32),
    ],
)


def _mm_mid_body(p_ref, y_ref, dinv_ref, b_ref, w_ref, out_ref):
    dinv = dinv_ref[...]
    seg = p_ref[0] + p_ref[1] + y_ref[...]
    h = jnp.maximum(seg * dinv + b_ref[...], 0.0)
    out_ref[...] = jnp.dot(h, w_ref[...],
                           preferred_element_type=jnp.float32) * dinv


_mm_mid = pl.pallas_call(
    _mm_mid_body,
    grid=(NPAD // BLK,),
    in_specs=[
        pl.BlockSpec((2, BLK, F), lambda i: (0, i, 0)),
        pl.BlockSpec((BLK, F), lambda i: (i, 0)),
        pl.BlockSpec((BLK, 1), lambda i: (i, 0)),
        pl.BlockSpec((1, F), lambda i: (0, 0)),
        pl.BlockSpec((F, F), lambda i: (0, 0)),
    ],
    out_specs=pl.BlockSpec((BLK, F), lambda i: (i, 0)),
    out_shape=jax.ShapeDtypeStruct((NPAD, F), jnp.float32),
)

FBLK = 2000


def _final_body(p_ref, y_ref, dinv_ref, b_ref, out_ref):
    seg = p_ref[0] + p_ref[1] + y_ref[...]
    out_ref[...] = seg * dinv_ref[...] + b_ref[...]


_final = pl.pallas_call(
    _final_body,
    grid=(N // FBLK,),
    in_specs=[
        pl.BlockSpec((2, FBLK, F), lambda i: (0, i, 0)),
        pl.BlockSpec((FBLK, F), lambda i: (i, 0)),
        pl.BlockSpec((FBLK, 1), lambda i: (i, 0)),
        pl.BlockSpec((1, F), lambda i: (0, 0)),
    ],
    out_specs=pl.BlockSpec((FBLK, F), lambda i: (i, 0)),
    out_shape=jax.ShapeDtypeStruct((N, F), jnp.float32),
)


def kernel(feature, edge_index, W1, b1, W2, b2, W3, b3):
    ei = edge_index.astype(jnp.int32)
    pad = N + jnp.arange(2 * (EPAD - E), dtype=jnp.int32) % (NPAD - N)
    sd = jnp.concatenate([ei, pad.reshape(2, EPAD - E)], axis=1)
    sd = sd.reshape(2, NW, NCHUNK, CHUNK)
    xpad = jnp.pad(feature, ((0, NPAD - N), (0, 0)))

    degt = _deg_kernel(sd).T

    y1, dinv = _mm_first(xpad, W1, degt)
    p = _agg_kernel(y1, sd)
    y2 = _mm_mid(p, y1, dinv, b1.reshape(1, F), W2)
    p = _agg_kernel(y2, sd)
    y3 = _mm_mid(p, y2, dinv, b2.reshape(1, F), W3)
    p = _agg_kernel(y3, sd)
    return _final(p, y3, dinv, b3.reshape(1, F))


# speedup vs baseline: 1.1076x; 1.1076x over previous
import functools

import jax
import jax.numpy as jnp
from jax import lax
from jax.experimental import pallas as pl
from jax.experimental.pallas import tpu as pltpu
from jax.experimental.pallas import tpu_sc as plsc

N = 10000
NPAD = 10240
F = 128
E = 320000
NC = 2
NS = 16
NW = NC * NS
CHUNK = 64
GC = 1
AGG_CHUNK = GC * CHUNK
NCHUNK = 160
PHASES = 4
PCH = NCHUNK // PHASES
EPT = NCHUNK * AGG_CHUNK
EPAD = EPT * NW
ROWS_PER_TILE = NPAD // NS

_sc_mesh = plsc.VectorSubcoreMesh(core_axis_name="c", subcore_axis_name="s")


@functools.partial(
    pl.kernel,
    out_type=jax.ShapeDtypeStruct((NW, NPAD), jnp.float32),
    mesh=_sc_mesh,
    scratch_types=[
        pltpu.VMEM((NCHUNK, CHUNK), jnp.int32),
        pltpu.VMEM((NPAD,), jnp.float32),
    ],
    compiler_params=pltpu.CompilerParams(needs_layout_passes=False),
)
def _deg_kernel(sd_hbm, out_hbm, didx_v, deg_v):
    c = lax.axis_index("c")
    s = lax.axis_index("s")
    w = c * NS + s
    zeros16 = jnp.zeros((16,), jnp.float32)

    @pl.loop(0, NPAD // 16)
    def _(i):
        deg_v[pl.ds(i * 16, 16)] = zeros16

    pltpu.sync_copy(sd_hbm.at[1, w], didx_v)
    ones16 = jnp.ones((16,), jnp.float32)

    @pl.loop(0, NCHUNK)
    def _(i):
        for j in range(CHUNK // 16):
            idx = didx_v[i, pl.ds(j * 16, 16)]
            plsc.addupdate_scatter(deg_v, [idx], ones16)

    pltpu.sync_copy(deg_v, out_hbm.at[w])


NBUF = 4


@functools.partial(
    pl.kernel,
    out_type=jax.ShapeDtypeStruct((NC, NPAD, F), jnp.float32),
    mesh=_sc_mesh,
    scratch_types=[
        pltpu.VMEM((PCH, CHUNK), jnp.int32),
        pltpu.VMEM((PCH, CHUNK), jnp.int32),
        pltpu.VMEM((AGG_CHUNK, F), jnp.float32),
        pltpu.VMEM((AGG_CHUNK, F), jnp.float32),
        pltpu.VMEM((AGG_CHUNK, F), jnp.float32),
        pltpu.VMEM((AGG_CHUNK, F), jnp.float32),
        pltpu.VMEM_SHARED((NPAD, F), jnp.float32),
        pltpu.SemaphoreType.DMA,
        pltpu.SemaphoreType.DMA,
        pltpu.SemaphoreType.DMA,
        pltpu.SemaphoreType.DMA,
        pltpu.SemaphoreType.DMA,
        pltpu.SemaphoreType.DMA,
        pltpu.SemaphoreType.DMA,
        pltpu.SemaphoreType.DMA,
    ],
)
def _agg_kernel(y_hbm, sd_hbm, out_hbm, sidx_v, didx_v, rows0_v,
                rows1_v, rows2_v, rows3_v, acc_sh, g0, g1, g2, g3, s0, s1,
                s2, s3):
    c = lax.axis_index("c")
    s = lax.axis_index("s")
    w = c * NS + s
    rows = (rows0_v, rows1_v, rows2_v, rows3_v)
    gsem = (g0, g1, g2, g3)
    ssem = (s0, s1, s2, s3)
    zeros16 = jnp.zeros((16,), jnp.float32)

    @pl.loop(0, AGG_CHUNK)
    def _(i):
        for j in range(F // 16):
            rows0_v[i, pl.ds(j * 16, 16)] = zeros16

    for r in range(ROWS_PER_TILE // AGG_CHUNK):
        pltpu.sync_copy(
            rows0_v.at[pl.ds(0, AGG_CHUNK)],
            acc_sh.at[pl.ds(s * ROWS_PER_TILE + r * AGG_CHUNK, AGG_CHUNK)])
    plsc.subcore_barrier()

    def gather(i, b):
        pltpu.async_copy(y_hbm.at[sidx_v.at[i]], rows[b], gsem[b])

    def wait_gather(i, b):
        pltpu.make_async_copy(y_hbm.at[sidx_v.at[i]], rows[b], gsem[b]).wait()

    def scatter(i, b):
        pltpu.async_copy(rows[b], acc_sh.at[didx_v.at[i]], ssem[b], add=True)

    def wait_scatter(i, b):
        pltpu.make_async_copy(rows[b], acc_sh.at[didx_v.at[i]],
                              ssem[b]).wait()

    for h in range(PHASES):
        pltpu.sync_copy(sd_hbm.at[0, w, pl.ds(h * PCH, PCH)], sidx_v)
        pltpu.sync_copy(sd_hbm.at[1, w, pl.ds(h * PCH, PCH)], didx_v)

        for b in range(NBUF):
            gather(b, b)

        @pl.loop(0, PCH, step=NBUF)
        def _(i):
            for b in range(NBUF):
                wait_gather(i + b, b)
                scatter(i + b, b)

                @pl.when(i + b + NBUF < PCH)
                def _():
                    wait_scatter(i + b, b)
                    gather(i + b + NBUF, b)

        for b in range(NBUF):
            wait_scatter(PCH - NBUF + b, b)

    plsc.subcore_barrier()
    pltpu.sync_copy(acc_sh.at[pl.ds(s * ROWS_PER_TILE, ROWS_PER_TILE)],
                    out_hbm.at[c, pl.ds(s * ROWS_PER_TILE, ROWS_PER_TILE)])


BLK = 1024


def _mm_first_body(x_ref, w_ref, degt_ref, y_ref, dinv_ref):
    deg = jnp.sum(degt_ref[...], axis=1, keepdims=True)
    dinv = lax.rsqrt(1.0 + deg)
    y_ref[...] = jnp.dot(x_ref[...], w_ref[...],
                         preferred_element_type=jnp.float32) * dinv
    dinv_ref[...] = dinv


_mm_first = pl.pallas_call(
    _mm_first_body,
    grid=(NPAD // BLK,),
    in_specs=[
        pl.BlockSpec((BLK, F), lambda i: (i, 0)),
        pl.BlockSpec((F, F), lambda i: (0, 0)),
        pl.BlockSpec((BLK, NW), lambda i: (i, 0)),
    ],
    out_specs=[
        pl.BlockSpec((BLK, F), lambda i: (i, 0)),
        pl.BlockSpec((BLK, 1), lambda i: (i, 0)),
    ],
    out_shape=[
        jax.ShapeDtypeStruct((NPAD, F), jnp.float32),
        jax.ShapeDtypeStruct((NPAD, 1), jnp.float32),
    ],
)


def _mm_mid_body(p_ref, y_ref, dinv_ref, b_ref, w_ref, out_ref):
    dinv = dinv_ref[...]
    seg = p_ref[0] + p_ref[1] + y_ref[...]
    h = jnp.maximum(seg * dinv + b_ref[...], 0.0)
    out_ref[...] = jnp.dot(h, w_ref[...],
                           preferred_element_type=jnp.float32) * dinv


_mm_mid = pl.pallas_call(
    _mm_mid_body,
    grid=(NPAD // BLK,),
    in_specs=[
        pl.BlockSpec((2, BLK, F), lambda i: (0, i, 0)),
        pl.BlockSpec((BLK, F), lambda i: (i, 0)),
        pl.BlockSpec((BLK, 1), lambda i: (i, 0)),
        pl.BlockSpec((1, F), lambda i: (0, 0)),
        pl.BlockSpec((F, F), lambda i: (0, 0)),
    ],
    out_specs=pl.BlockSpec((BLK, F), lambda i: (i, 0)),
    out_shape=jax.ShapeDtypeStruct((NPAD, F), jnp.float32),
)

FBLK = 2000


def _final_body(p_ref, y_ref, dinv_ref, b_ref, out_ref):
    seg = p_ref[0] + p_ref[1] + y_ref[...]
    out_ref[...] = seg * dinv_ref[...] + b_ref[...]


_final = pl.pallas_call(
    _final_body,
    grid=(N // FBLK,),
    in_specs=[
        pl.BlockSpec((2, FBLK, F), lambda i: (0, i, 0)),
        pl.BlockSpec((FBLK, F), lambda i: (i, 0)),
        pl.BlockSpec((FBLK, 1), lambda i: (i, 0)),
        pl.BlockSpec((1, F), lambda i: (0, 0)),
    ],
    out_specs=pl.BlockSpec((FBLK, F), lambda i: (i, 0)),
    out_shape=jax.ShapeDtypeStruct((N, F), jnp.float32),
)


def kernel(feature, edge_index, W1, b1, W2, b2, W3, b3):
    ei = edge_index.astype(jnp.int32)
    pad = N + jnp.arange(2 * (EPAD - E), dtype=jnp.int32) % (NPAD - N)
    sd = jnp.concatenate([ei, pad.reshape(2, EPAD - E)], axis=1)
    sd = sd.reshape(2, NW, NCHUNK, CHUNK)
    xpad = jnp.pad(feature, ((0, NPAD - N), (0, 0)))

    degt = _deg_kernel(sd).T

    y1, dinv = _mm_first(xpad, W1, degt)
    p = _agg_kernel(y1, sd)
    y2 = _mm_mid(p, y1, dinv, b1.reshape(1, F), W2)
    p = _agg_kernel(y2, sd)
    y3 = _mm_mid(p, y2, dinv, b2.reshape(1, F), W3)
    p = _agg_kernel(y3, sd)
    return _final(p, y3, dinv, b3.reshape(1, F))


# speedup vs baseline: 1.1194x; 1.0107x over previous
import functools

import jax
import jax.numpy as jnp
from jax import lax
from jax.experimental import pallas as pl
from jax.experimental.pallas import tpu as pltpu
from jax.experimental.pallas import tpu_sc as plsc

N = 10000
NPAD = 10240
F = 128
E = 320000
NC = 2
NS = 16
NW = NC * NS
CHUNK = 64
GC = 1
AGG_CHUNK = GC * CHUNK
NCHUNK = 160
PHASES = 4
PCH = NCHUNK // PHASES
EPT = NCHUNK * AGG_CHUNK
EPAD = EPT * NW
ROWS_PER_TILE = NPAD // NS

_sc_mesh = plsc.VectorSubcoreMesh(core_axis_name="c", subcore_axis_name="s")


@functools.partial(
    pl.kernel,
    out_type=jax.ShapeDtypeStruct((NW, NPAD), jnp.float32),
    mesh=_sc_mesh,
    scratch_types=[
        pltpu.VMEM((NCHUNK, CHUNK), jnp.int32),
        pltpu.VMEM((NPAD,), jnp.float32),
    ],
    compiler_params=pltpu.CompilerParams(needs_layout_passes=False),
)
def _deg_kernel(sd_hbm, out_hbm, didx_v, deg_v):
    c = lax.axis_index("c")
    s = lax.axis_index("s")
    w = c * NS + s
    zeros16 = jnp.zeros((16,), jnp.float32)

    @pl.loop(0, NPAD // 16)
    def _(i):
        deg_v[pl.ds(i * 16, 16)] = zeros16

    pltpu.sync_copy(sd_hbm.at[1, w], didx_v)
    ones16 = jnp.ones((16,), jnp.float32)

    @pl.loop(0, NCHUNK)
    def _(i):
        for j in range(CHUNK // 16):
            idx = didx_v[i, pl.ds(j * 16, 16)]
            plsc.addupdate_scatter(deg_v, [idx], ones16)

    pltpu.sync_copy(deg_v, out_hbm.at[w])


NBUF = 4


@functools.partial(
    pl.kernel,
    out_type=jax.ShapeDtypeStruct((NC, NPAD, F), jnp.float32),
    mesh=_sc_mesh,
    scratch_types=[
        pltpu.VMEM((PCH, CHUNK), jnp.int32),
        pltpu.VMEM((PCH, CHUNK), jnp.int32),
        pltpu.VMEM((AGG_CHUNK, F), jnp.float32),
        pltpu.VMEM((AGG_CHUNK, F), jnp.float32),
        pltpu.VMEM((AGG_CHUNK, F), jnp.float32),
        pltpu.VMEM((AGG_CHUNK, F), jnp.float32),
        pltpu.VMEM_SHARED((NPAD, F), jnp.float32),
        pltpu.SemaphoreType.DMA,
        pltpu.SemaphoreType.DMA,
        pltpu.SemaphoreType.DMA,
        pltpu.SemaphoreType.DMA,
        pltpu.SemaphoreType.DMA,
        pltpu.SemaphoreType.DMA,
        pltpu.SemaphoreType.DMA,
        pltpu.SemaphoreType.DMA,
    ],
)
def _agg_kernel(y_hbm, sd_hbm, out_hbm, sidx_v, didx_v, rows0_v,
                rows1_v, rows2_v, rows3_v, acc_sh, g0, g1, g2, g3, s0, s1,
                s2, s3):
    c = lax.axis_index("c")
    s = lax.axis_index("s")
    w = c * NS + s
    rows = (rows0_v, rows1_v, rows2_v, rows3_v)
    gsem = (g0, g1, g2, g3)
    ssem = (s0, s1, s2, s3)
    zeros16 = jnp.zeros((16,), jnp.float32)

    @pl.loop(0, AGG_CHUNK)
    def _(i):
        for j in range(F // 16):
            rows0_v[i, pl.ds(j * 16, 16)] = zeros16

    for r in range(ROWS_PER_TILE // AGG_CHUNK):
        pltpu.sync_copy(
            rows0_v.at[pl.ds(0, AGG_CHUNK)],
            acc_sh.at[pl.ds(s * ROWS_PER_TILE + r * AGG_CHUNK, AGG_CHUNK)])
    plsc.subcore_barrier()

    def gather(i, b):
        pltpu.async_copy(y_hbm.at[sidx_v.at[i]], rows[b], gsem[b])

    def wait_gather(i, b):
        pltpu.make_async_copy(y_hbm.at[sidx_v.at[i]], rows[b], gsem[b]).wait()

    def scatter(i, b):
        pltpu.async_copy(rows[b], acc_sh.at[didx_v.at[i]], ssem[b], add=True)

    def wait_scatter(i, b):
        pltpu.make_async_copy(rows[b], acc_sh.at[didx_v.at[i]],
                              ssem[b]).wait()

    for h in range(PHASES):
        pltpu.sync_copy(sd_hbm.at[0, w, pl.ds(h * PCH, PCH)], sidx_v)
        pltpu.sync_copy(sd_hbm.at[1, w, pl.ds(h * PCH, PCH)], didx_v)

        for b in range(NBUF):
            gather(b, b)

        @pl.loop(0, PCH, step=NBUF)
        def _(i):
            for b in range(NBUF):
                wait_gather(i + b, b)
                scatter(i + b, b)

                @pl.when(i + b + NBUF < PCH)
                def _():
                    wait_scatter(i + b, b)
                    gather(i + b + NBUF, b)

        for b in range(NBUF):
            wait_scatter(PCH - NBUF + b, b)

    plsc.subcore_barrier()
    pltpu.sync_copy(acc_sh.at[pl.ds(s * ROWS_PER_TILE, ROWS_PER_TILE)],
                    out_hbm.at[c, pl.ds(s * ROWS_PER_TILE, ROWS_PER_TILE)])


BLK = 1024


def _mm_first_body(x_ref, w_ref, degt_ref, y_ref, dinv_ref):
    ones = jnp.ones((NW, 1), jnp.float32)
    deg = lax.dot_general(degt_ref[...], ones, (((0,), (0,)), ((), ())),
                          preferred_element_type=jnp.float32)
    dinv = lax.rsqrt(1.0 + deg)
    y_ref[...] = jnp.dot(x_ref[...], w_ref[...],
                         preferred_element_type=jnp.float32) * dinv
    dinv_ref[...] = dinv


_mm_first = pl.pallas_call(
    _mm_first_body,
    grid=(NPAD // BLK,),
    in_specs=[
        pl.BlockSpec((BLK, F), lambda i: (i, 0)),
        pl.BlockSpec((F, F), lambda i: (0, 0)),
        pl.BlockSpec((NW, BLK), lambda i: (0, i)),
    ],
    out_specs=[
        pl.BlockSpec((BLK, F), lambda i: (i, 0)),
        pl.BlockSpec((BLK, 1), lambda i: (i, 0)),
    ],
    out_shape=[
        jax.ShapeDtypeStruct((NPAD, F), jnp.float32),
        jax.ShapeDtypeStruct((NPAD, 1), jnp.float32),
    ],
)


def _mm_mid_body(p_ref, y_ref, dinv_ref, b_ref, w_ref, out_ref):
    dinv = dinv_ref[...]
    seg = p_ref[0] + p_ref[1] + y_ref[...]
    h = jnp.maximum(seg * dinv + b_ref[...], 0.0)
    out_ref[...] = jnp.dot(h, w_ref[...],
                           preferred_element_type=jnp.float32) * dinv


_mm_mid = pl.pallas_call(
    _mm_mid_body,
    grid=(NPAD // BLK,),
    in_specs=[
        pl.BlockSpec((2, BLK, F), lambda i: (0, i, 0)),
        pl.BlockSpec((BLK, F), lambda i: (i, 0)),
        pl.BlockSpec((BLK, 1), lambda i: (i, 0)),
        pl.BlockSpec((1, F), lambda i: (0, 0)),
        pl.BlockSpec((F, F), lambda i: (0, 0)),
    ],
    out_specs=pl.BlockSpec((BLK, F), lambda i: (i, 0)),
    out_shape=jax.ShapeDtypeStruct((NPAD, F), jnp.float32),
)

FBLK = 2000


def _final_body(p_ref, y_ref, dinv_ref, b_ref, out_ref):
    seg = p_ref[0] + p_ref[1] + y_ref[...]
    out_ref[...] = seg * dinv_ref[...] + b_ref[...]


_final = pl.pallas_call(
    _final_body,
    grid=(N // FBLK,),
    in_specs=[
        pl.BlockSpec((2, FBLK, F), lambda i: (0, i, 0)),
        pl.BlockSpec((FBLK, F), lambda i: (i, 0)),
        pl.BlockSpec((FBLK, 1), lambda i: (i, 0)),
        pl.BlockSpec((1, F), lambda i: (0, 0)),
    ],
    out_specs=pl.BlockSpec((FBLK, F), lambda i: (i, 0)),
    out_shape=jax.ShapeDtypeStruct((N, F), jnp.float32),
)


def kernel(feature, edge_index, W1, b1, W2, b2, W3, b3):
    ei = edge_index.astype(jnp.int32)
    pad = N + jnp.arange(2 * (EPAD - E), dtype=jnp.int32) % (NPAD - N)
    sd = jnp.concatenate([ei, pad.reshape(2, EPAD - E)], axis=1)
    sd = sd.reshape(2, NW, NCHUNK, CHUNK)
    xpad = jnp.pad(feature, ((0, NPAD - N), (0, 0)))

    degt = _deg_kernel(sd)

    y1, dinv = _mm_first(xpad, W1, degt)
    p = _agg_kernel(y1, sd)
    y2 = _mm_mid(p, y1, dinv, b1.reshape(1, F), W2)
    p = _agg_kernel(y2, sd)
    y3 = _mm_mid(p, y2, dinv, b2.reshape(1, F), W3)
    p = _agg_kernel(y3, sd)
    return _final(p, y3, dinv, b3.reshape(1, F))
